# baseline (speedup 1.0000x reference)
import functools

import jax
import jax.numpy as jnp
from jax.experimental import pallas as pl
from jax.experimental.pallas import tpu as pltpu

SB = 128
D = 1024
S = 8192
B = 4


def _body(x_ref, m_ref, o_ref, carry_ref):
    j = pl.program_id(1)

    @pl.when(j == 0)
    def _():
        carry_ref[...] = jnp.zeros_like(carry_ref)

    xm = jnp.where(m_ref[0], x_ref[0], 0.0)
    row = jax.lax.broadcasted_iota(jnp.int32, (SB, SB), 0)
    col = jax.lax.broadcasted_iota(jnp.int32, (SB, SB), 1)
    tri = (row >= col).astype(jnp.float32)
    acc = jax.lax.dot(tri, xm, preferred_element_type=jnp.float32)
    out = acc + carry_ref[...]
    o_ref[...] = out[None]
    carry_ref[...] = out[-1:, :]


@jax.jit
def kernel(x, mask):
    grid = (B, S // SB)
    return pl.pallas_call(
        _body,
        grid=grid,
        in_specs=[
            pl.BlockSpec((1, SB, D), lambda b, j: (b, j, 0)),
            pl.BlockSpec((1, SB, D), lambda b, j: (b, j, 0)),
        ],
        out_specs=pl.BlockSpec((1, SB, D), lambda b, j: (b, j, 0)),
        out_shape=jax.ShapeDtypeStruct((B, S, D), jnp.float32),
        scratch_shapes=[pltpu.VMEM((1, D), jnp.float32)],
        compiler_params=pltpu.CompilerParams(
            dimension_semantics=("arbitrary", "arbitrary"),
        ),
    )(x, mask)


# speedup vs baseline: 1.5458x; 1.5458x over previous
import functools

import jax
import jax.numpy as jnp
from jax.experimental import pallas as pl
from jax.experimental.pallas import tpu as pltpu

SB = 512
D = 1024
S = 8192
B = 4


def _body(x_ref, m_ref, o_ref, carry_ref):
    j = pl.program_id(1)

    @pl.when(j == 0)
    def _():
        carry_ref[...] = jnp.zeros_like(carry_ref)

    xm = jnp.where(m_ref[0], x_ref[0], 0.0)
    row = jax.lax.broadcasted_iota(jnp.int32, (SB, SB), 0)
    col = jax.lax.broadcasted_iota(jnp.int32, (SB, SB), 1)
    tri = (row >= col).astype(jnp.float32)
    acc = jax.lax.dot(tri, xm, preferred_element_type=jnp.float32)
    out = acc + carry_ref[...]
    o_ref[...] = out[None]
    carry_ref[...] = out[-1:, :]


@jax.jit
def kernel(x, mask):
    grid = (B, S // SB)
    return pl.pallas_call(
        _body,
        grid=grid,
        in_specs=[
            pl.BlockSpec((1, SB, D), lambda b, j: (b, j, 0)),
            pl.BlockSpec((1, SB, D), lambda b, j: (b, j, 0)),
        ],
        out_specs=pl.BlockSpec((1, SB, D), lambda b, j: (b, j, 0)),
        out_shape=jax.ShapeDtypeStruct((B, S, D), jnp.float32),
        scratch_shapes=[pltpu.VMEM((1, D), jnp.float32)],
        compiler_params=pltpu.CompilerParams(
            dimension_semantics=("arbitrary", "arbitrary"),
        ),
    )(x, mask)


# speedup vs baseline: 1.5734x; 1.0178x over previous
import functools

import jax
import jax.numpy as jnp
from jax.experimental import pallas as pl
from jax.experimental.pallas import tpu as pltpu

SB = 512
G = 256
D = 1024
S = 8192
B = 4


def _body(x_ref, m_ref, o_ref, carry_ref):
    j = pl.program_id(1)

    @pl.when(j == 0)
    def _():
        carry_ref[...] = jnp.zeros_like(carry_ref)

    xm = jnp.where(m_ref[0], x_ref[0], 0.0)
    row = jax.lax.broadcasted_iota(jnp.int32, (G, G), 0)
    col = jax.lax.broadcasted_iota(jnp.int32, (G, G), 1)
    tri = (row >= col).astype(jnp.float32)
    offs = carry_ref[...]
    for c in range(SB // G):
        p = jax.lax.dot(tri, xm[c * G:(c + 1) * G], preferred_element_type=jnp.float32)
        out = p + offs
        o_ref[0, c * G:(c + 1) * G, :] = out
        offs = out[-1:, :]
    carry_ref[...] = offs


@jax.jit
def kernel(x, mask):
    grid = (B, S // SB)
    return pl.pallas_call(
        _body,
        grid=grid,
        in_specs=[
            pl.BlockSpec((1, SB, D), lambda b, j: (b, j, 0)),
            pl.BlockSpec((1, SB, D), lambda b, j: (b, j, 0)),
        ],
        out_specs=pl.BlockSpec((1, SB, D), lambda b, j: (b, j, 0)),
        out_shape=jax.ShapeDtypeStruct((B, S, D), jnp.float32),
        scratch_shapes=[pltpu.VMEM((1, D), jnp.float32)],
        compiler_params=pltpu.CompilerParams(
            dimension_semantics=("arbitrary", "arbitrary"),
        ),
    )(x, mask)


# speedup vs baseline: 2.4325x; 1.5460x over previous
import jax
import jax.numpy as jnp
from jax.experimental import pallas as pl
from jax.experimental.pallas import tpu as pltpu

B = 4
S = 8192
D = 1024
R = 512
G = 256
N = 8
NUM = (B * S) // R
PER_BATCH = S // R


def _body(x_hbm, m_hbm, o_hbm, xbuf, mbuf, obuf, lsx, lsm, ssem):
    def start_load(i):
        slot = i % N
        pltpu.make_async_copy(
            x_hbm.at[pl.ds(i * R, R), :], xbuf.at[slot], lsx.at[slot]).start()
        pltpu.make_async_copy(
            m_hbm.at[pl.ds(i * R, R), :], mbuf.at[slot], lsm.at[slot]).start()

    row = jax.lax.broadcasted_iota(jnp.int32, (G, G), 0)
    col = jax.lax.broadcasted_iota(jnp.int32, (G, G), 1)
    tri = (row >= col).astype(jnp.float32)

    for i in range(N):
        start_load(i)

    carry = jnp.zeros((1, D), jnp.float32)
    for i in range(NUM):
        slot = i % N
        pltpu.make_async_copy(
            x_hbm.at[pl.ds(i * R, R), :], xbuf.at[slot], lsx.at[slot]).wait()
        pltpu.make_async_copy(
            m_hbm.at[pl.ds(i * R, R), :], mbuf.at[slot], lsm.at[slot]).wait()
        if i >= N:
            pltpu.make_async_copy(
                obuf.at[slot], o_hbm.at[pl.ds((i - N) * R, R), :],
                ssem.at[slot]).wait()
        if i % PER_BATCH == 0:
            carry = jnp.zeros((1, D), jnp.float32)
        xm = jnp.where(mbuf[slot] != 0, xbuf[slot], 0.0)
        for c in range(R // G):
            p = jax.lax.dot(tri, xm[c * G:(c + 1) * G],
                            preferred_element_type=jnp.float32)
            outc = p + carry
            obuf[slot, c * G:(c + 1) * G, :] = outc
            carry = outc[-1:, :]
        pltpu.make_async_copy(
            obuf.at[slot], o_hbm.at[pl.ds(i * R, R), :], ssem.at[slot]).start()
        if i + N < NUM:
            start_load(i + N)

    for i in range(NUM - N, NUM):
        slot = i % N
        pltpu.make_async_copy(
            obuf.at[slot], o_hbm.at[pl.ds(i * R, R), :], ssem.at[slot]).wait()


@jax.jit
def kernel(x, mask):
    xf = x.reshape(B * S, D)
    mf = mask.reshape(B * S, D).view(jnp.int8)
    out = pl.pallas_call(
        _body,
        in_specs=[
            pl.BlockSpec(memory_space=pltpu.MemorySpace.HBM),
            pl.BlockSpec(memory_space=pltpu.MemorySpace.HBM),
        ],
        out_specs=pl.BlockSpec(memory_space=pltpu.MemorySpace.HBM),
        out_shape=jax.ShapeDtypeStruct((B * S, D), jnp.float32),
        scratch_shapes=[
            pltpu.VMEM((N, R, D), jnp.float32),
            pltpu.VMEM((N, R, D), jnp.int8),
            pltpu.VMEM((N, R, D), jnp.float32),
            pltpu.SemaphoreType.DMA((N,)),
            pltpu.SemaphoreType.DMA((N,)),
            pltpu.SemaphoreType.DMA((N,)),
        ],
    )(xf, mf)
    return out.reshape(B, S, D)


# speedup vs baseline: 2.4532x; 1.0085x over previous
import jax
import jax.numpy as jnp
from jax.experimental import pallas as pl
from jax.experimental.pallas import tpu as pltpu

B = 4
S = 8192
D = 1024
R = 256
G = 256
N = 16
NUM = (B * S) // R
PER_BATCH = S // R


def _body(x_hbm, m_hbm, o_hbm, xbuf, mbuf, obuf, lsx, lsm, ssem):
    def start_load(i):
        slot = i % N
        pltpu.make_async_copy(
            x_hbm.at[pl.ds(i * R, R), :], xbuf.at[slot], lsx.at[slot]).start()
        pltpu.make_async_copy(
            m_hbm.at[pl.ds(i * R, R), :], mbuf.at[slot], lsm.at[slot]).start()

    row = jax.lax.broadcasted_iota(jnp.int32, (G, G), 0)
    col = jax.lax.broadcasted_iota(jnp.int32, (G, G), 1)
    tri = (row >= col).astype(jnp.float32)

    for i in range(N):
        start_load(i)

    carry = jnp.zeros((1, D), jnp.float32)
    for i in range(NUM):
        slot = i % N
        pltpu.make_async_copy(
            x_hbm.at[pl.ds(i * R, R), :], xbuf.at[slot], lsx.at[slot]).wait()
        pltpu.make_async_copy(
            m_hbm.at[pl.ds(i * R, R), :], mbuf.at[slot], lsm.at[slot]).wait()
        if i >= N:
            pltpu.make_async_copy(
                obuf.at[slot], o_hbm.at[pl.ds((i - N) * R, R), :],
                ssem.at[slot]).wait()
        if i % PER_BATCH == 0:
            carry = jnp.zeros((1, D), jnp.float32)
        xm = jnp.where(mbuf[slot] != 0, xbuf[slot], 0.0)
        for c in range(R // G):
            p = jax.lax.dot(tri, xm[c * G:(c + 1) * G],
                            preferred_element_type=jnp.float32)
            outc = p + carry
            obuf[slot, c * G:(c + 1) * G, :] = outc
            carry = outc[-1:, :]
        pltpu.make_async_copy(
            obuf.at[slot], o_hbm.at[pl.ds(i * R, R), :], ssem.at[slot]).start()
        if i + N < NUM:
            start_load(i + N)

    for i in range(NUM - N, NUM):
        slot = i % N
        pltpu.make_async_copy(
            obuf.at[slot], o_hbm.at[pl.ds(i * R, R), :], ssem.at[slot]).wait()


@jax.jit
def kernel(x, mask):
    xf = x.reshape(B * S, D)
    mf = mask.reshape(B * S, D).view(jnp.int8)
    out = pl.pallas_call(
        _body,
        in_specs=[
            pl.BlockSpec(memory_space=pltpu.MemorySpace.HBM),
            pl.BlockSpec(memory_space=pltpu.MemorySpace.HBM),
        ],
        out_specs=pl.BlockSpec(memory_space=pltpu.MemorySpace.HBM),
        out_shape=jax.ShapeDtypeStruct((B * S, D), jnp.float32),
        scratch_shapes=[
            pltpu.VMEM((N, R, D), jnp.float32),
            pltpu.VMEM((N, R, D), jnp.int8),
            pltpu.VMEM((N, R, D), jnp.float32),
            pltpu.SemaphoreType.DMA((N,)),
            pltpu.SemaphoreType.DMA((N,)),
            pltpu.SemaphoreType.DMA((N,)),
        ],
    )(xf, mf)
    return out.reshape(B, S, D)
